# baseline (speedup 1.0000x reference)
import functools

import jax
import jax.numpy as jnp
from jax import lax
from jax.experimental import pallas as pl
from jax.experimental.pallas import tpu as pltpu
from jax.experimental.pallas import tpu_sc as plsc

VOCAB = 1000000
EMBED_DIM = 64
BATCH = 4096
HIST = 200

NUM_CORES = 2
NUM_SUBCORES = 16
NUM_WORKERS = NUM_CORES * NUM_SUBCORES

B_TOTAL = BATCH * HIST
B_PER_W = B_TOTAL // NUM_WORKERS
ROWS_PER_W = BATCH // NUM_WORKERS
CHUNK = HIST
N_CHUNKS = ROWS_PER_W
NBUF = 4


def _make_gather():
  mesh = plsc.VectorSubcoreMesh(
      core_axis_name="c", subcore_axis_name="s",
      num_cores=NUM_CORES, num_subcores=NUM_SUBCORES)

  @functools.partial(
      pl.kernel,
      mesh=mesh,
      out_type=jax.ShapeDtypeStruct((BATCH, HIST, EMBED_DIM), jnp.float32),
      scratch_types=[
          pltpu.VMEM((B_PER_W,), jnp.int32),
          pltpu.VMEM((NBUF, CHUNK, EMBED_DIM), jnp.float32),
          [pltpu.SemaphoreType.DMA] * NBUF,
          [pltpu.SemaphoreType.DMA] * NBUF,
      ],
      compiler_params=pltpu.CompilerParams(use_tc_tiling_on_sc=False,
                                           needs_layout_passes=True),
  )
  def gather_kernel(idx_hbm, table_hbm, out_hbm, idx_v, rows_v, gsems, ssems):
    wid = lax.axis_index("s") * NUM_CORES + lax.axis_index("c")
    base = wid * B_PER_W
    row_base = wid * ROWS_PER_W
    pltpu.sync_copy(idx_hbm.at[pl.ds(base, B_PER_W)], idx_v)

    @pl.loop(0, N_CHUNKS, step=NBUF)
    def _group(g0):
      for b in range(NBUF):
        g = g0 + b

        @pl.when(g0 > 0)
        def _():
          pltpu.make_async_copy(
              rows_v.at[b], out_hbm.at[0], ssems[b]).wait()

        pltpu.async_copy(
            table_hbm.at[idx_v.at[pl.ds(g * CHUNK, CHUNK)]],
            rows_v.at[b], gsems[b])
      for b in range(NBUF):
        g = g0 + b
        pltpu.make_async_copy(
            table_hbm.at[pl.ds(0, CHUNK)], rows_v.at[b], gsems[b]).wait()
        pltpu.async_copy(
            rows_v.at[b], out_hbm.at[row_base + g], ssems[b])

    for b in range(NBUF):
      pltpu.make_async_copy(
          rows_v.at[b], out_hbm.at[0], ssems[b]).wait()

  return gather_kernel


_gather = _make_gather()


@jax.jit
def kernel(token_ids, weight):
  idx = token_ids.reshape(-1).astype(jnp.int32)
  return _gather(idx, weight)


# speedup vs baseline: 1.2232x; 1.2232x over previous
import functools

import jax
import jax.numpy as jnp
from jax import lax
from jax.experimental import pallas as pl
from jax.experimental.pallas import tpu as pltpu
from jax.experimental.pallas import tpu_sc as plsc

VOCAB = 1000000
EMBED_DIM = 64
PAD_DIM = 128
BATCH = 4096
HIST = 200

NUM_CORES = 2
NUM_SUBCORES = 16
NUM_WORKERS = NUM_CORES * NUM_SUBCORES

B_TOTAL = BATCH * HIST
B_PER_W = B_TOTAL // NUM_WORKERS
ROWS_PER_W = BATCH // NUM_WORKERS
CHUNK = HIST
N_CHUNKS = ROWS_PER_W
NBUF = 4


def _make_gather():
  mesh = plsc.VectorSubcoreMesh(
      core_axis_name="c", subcore_axis_name="s",
      num_cores=NUM_CORES, num_subcores=NUM_SUBCORES)

  @functools.partial(
      pl.kernel,
      mesh=mesh,
      out_type=jax.ShapeDtypeStruct((BATCH, HIST, PAD_DIM), jnp.float32),
      scratch_types=[
          pltpu.VMEM((B_PER_W,), jnp.int32),
          pltpu.VMEM((NBUF, CHUNK, PAD_DIM), jnp.float32),
          [pltpu.SemaphoreType.DMA] * NBUF,
          [pltpu.SemaphoreType.DMA] * NBUF,
      ],
      compiler_params=pltpu.CompilerParams(use_tc_tiling_on_sc=True,
                                           needs_layout_passes=True),
  )
  def gather_kernel(idx_hbm, table_hbm, out_hbm, idx_v, rows_v, gsems, ssems):
    wid = lax.axis_index("s") * NUM_CORES + lax.axis_index("c")
    base = wid * B_PER_W
    row_base = wid * ROWS_PER_W
    pltpu.sync_copy(idx_hbm.at[pl.ds(base, B_PER_W)], idx_v)

    @pl.loop(0, N_CHUNKS, step=NBUF)
    def _group(g0):
      for b in range(NBUF):
        g = g0 + b

        @pl.when(g0 > 0)
        def _():
          pltpu.make_async_copy(
              rows_v.at[b], out_hbm.at[0], ssems[b]).wait()

        pltpu.async_copy(
            table_hbm.at[idx_v.at[pl.ds(g * CHUNK, CHUNK)]],
            rows_v.at[b], gsems[b])
      for b in range(NBUF):
        g = g0 + b
        pltpu.make_async_copy(
            table_hbm.at[pl.ds(0, CHUNK)], rows_v.at[b], gsems[b]).wait()
        pltpu.async_copy(
            rows_v.at[b], out_hbm.at[row_base + g], ssems[b])

    for b in range(NBUF):
      pltpu.make_async_copy(
          rows_v.at[b], out_hbm.at[0], ssems[b]).wait()

  return gather_kernel


_gather = _make_gather()


@jax.jit
def kernel(token_ids, weight):
  idx = token_ids.reshape(-1).astype(jnp.int32)
  table = jnp.pad(weight, ((0, 0), (0, PAD_DIM - EMBED_DIM)))
  return _gather(idx, table)[..., :EMBED_DIM]
